# baseline (speedup 1.0000x reference)
import functools

import jax
import jax.numpy as jnp
from jax import lax
from jax.experimental import pallas as pl
from jax.experimental.pallas import tpu as pltpu
from jax.experimental.pallas import tpu_sc as plsc

N = 10000
NPAD = 10240
E = 320000
H = 128
R = 64
NB = 4
RCUT = 6.0
GAMMA = 10.0 / (RCUT * RCUT)

NW = 32
EPW = E // NW
CH = 80
NCH = EPW // CH
EBLK = 2048
TBLN = 32768
TSCALE = (TBLN - 2) / RCUT
NBLK = 1024

_mesh = plsc.VectorSubcoreMesh(core_axis_name="c", subcore_axis_name="s")


@functools.partial(
    pl.kernel,
    mesh=_mesh,
    out_type=[
        jax.ShapeDtypeStruct((NPAD, H), jnp.float32),
        jax.ShapeDtypeStruct((E,), jnp.float32),
    ],
    scratch_types=[
        pltpu.VMEM((64,), jnp.int32),
        pltpu.VMEM((64, H), jnp.float32),
        pltpu.VMEM((CH,), jnp.int32),
        pltpu.VMEM((CH,), jnp.int32),
        pltpu.VMEM((CH, H), jnp.float32),
        pltpu.VMEM((CH, H), jnp.float32),
        pltpu.VMEM((CH,), jnp.int32),
        pltpu.VMEM((CH,), jnp.int32),
        pltpu.VMEM((CH, H), jnp.float32),
        pltpu.VMEM((CH, H), jnp.float32),
        pltpu.VMEM((EPW,), jnp.float32),
        pltpu.SemaphoreType.DMA,
        pltpu.SemaphoreType.DMA,
        pltpu.SemaphoreType.DMA,
    ],
)
def _sc_prep(z_hbm, src_hbm, dst_hbm, posf_hbm, emb_hbm,
             h0_hbm, knot_hbm,
             zv, rowsv, sv0, dv0, prs0, prd0, sv1, dv1, prs1, prd1,
             kall, sem, semi0, semi1):
    c = lax.axis_index("c")
    s = lax.axis_index("s")
    wid = c * 16 + s
    bufs = ((sv0, dv0, prs0, prd0, semi0),
            (sv1, dv1, prs1, prd1, semi1))

    def emb_chunk(g, carry):
        rb = wid * (NPAD // NW) + g * 64
        pltpu.sync_copy(z_hbm.at[pl.ds(rb, 64)], zv)
        pltpu.async_copy(emb_hbm.at[zv], rowsv, sem).wait()
        pltpu.sync_copy(rowsv, h0_hbm.at[pl.ds(rb, 64)])
        return carry

    lax.fori_loop(0, (NPAD // NW) // 64, emb_chunk, 0)

    def idx_copies(g, b):
        sv, dv, semi = bufs[b][0], bufs[b][1], bufs[b][4]
        base = wid * EPW + g * CH
        return (pltpu.make_async_copy(src_hbm.at[pl.ds(base, CH)], sv, semi),
                pltpu.make_async_copy(dst_hbm.at[pl.ds(base, CH)], dv, semi))

    def pos_copies(b):
        sv, dv, prs, prd = bufs[b][0], bufs[b][1], bufs[b][2], bufs[b][3]
        return (pltpu.make_async_copy(posf_hbm.at[sv], prs, sem),
                pltpu.make_async_copy(posf_hbm.at[dv], prd, sem))

    lane = lax.iota(jnp.int32, 16)

    def phase(g, b, nxt_gather, nxt_idx):
        nb = 1 - b
        if nxt_gather:
            for cp in idx_copies(g + 1, nb):
                cp.wait()
            for cp in pos_copies(nb):
                cp.start()
        for cp in pos_copies(b):
            cp.wait()
        prs, prd = bufs[b][2], bufs[b][3]

        def dist16(k, c2):
            acc = jnp.zeros((16,), jnp.float32)
            for j in range(16):
                e = k * 16 + j
                df = prs[e, pl.ds(0, 16)] - prd[e, pl.ds(0, 16)]
                sq = df * df
                acc = jnp.where(lane == j, sq[0] + sq[1] + sq[2], acc)
            kall[pl.ds(g * CH + k * 16, 16)] = acc
            return c2

        lax.fori_loop(0, CH // 16, dist16, 0)
        if nxt_idx:
            for cp in idx_copies(g + 2, b):
                cp.start()

    for cp in idx_copies(0, 0) + idx_copies(1, 1):
        cp.start()
    for cp in idx_copies(0, 0):
        cp.wait()
    for cp in pos_copies(0):
        cp.start()

    def pair(k, carry):
        phase(2 * k, 0, True, True)
        phase(2 * k + 1, 1, True, True)
        return carry

    lax.fori_loop(0, (NCH - 3) // 2, pair, 0)
    phase(NCH - 3, 0, True, True)
    phase(NCH - 2, 1, True, False)
    phase(NCH - 1, 0, False, False)
    pltpu.sync_copy(kall, knot_hbm.at[pl.ds(wid * EPW, EPW)])


def _make_sc_scatter(blk):
    @functools.partial(
        pl.kernel,
        mesh=_mesh,
        out_type=jax.ShapeDtypeStruct((2, NPAD, H), jnp.float32),
        scratch_types=[
            pltpu.VMEM((CH,), jnp.int32),
            pltpu.VMEM((CH,), jnp.int32),
            pltpu.VMEM((CH,), jnp.int32),
            pltpu.VMEM((CH,), jnp.int32),
            pltpu.VMEM((CH, H), jnp.float32),
            pltpu.VMEM((CH, H), jnp.float32),
            pltpu.VMEM((CH,), jnp.int32),
            pltpu.VMEM((CH,), jnp.int32),
            pltpu.VMEM((CH,), jnp.int32),
            pltpu.VMEM((CH,), jnp.int32),
            pltpu.VMEM((CH, H), jnp.float32),
            pltpu.VMEM((CH, H), jnp.float32),
            pltpu.VMEM((CH,), jnp.int32),
            pltpu.VMEM((CH,), jnp.int32),
            pltpu.VMEM_SHARED((NPAD, H), jnp.float32),
            pltpu.SemaphoreType.DMA,
            pltpu.SemaphoreType.DMA,
            pltpu.SemaphoreType.DMA,
            pltpu.SemaphoreType.DMA,
            pltpu.SemaphoreType.DMA,
            pltpu.SemaphoreType.DMA,
            pltpu.SemaphoreType.DMA,
            pltpu.SemaphoreType.DMA,
        ],
    )
    def _sc_scatter(src_hbm, dst_hbm, knot_hbm, tbl_hbm, hl_hbm,
                    out_hbm,
                    sv0, dv0, iv0, ib0, rv0, wv0,
                    sv1, dv1, iv1, ib1, rv1, wv1, dsc0, dsc1, agg_sh,
                    semi0, semg0, semw0, sems0, semi1, semg1, semw1, sems1):
        c = lax.axis_index("c")
        s = lax.axis_index("s")
        wid = c * 16 + s
        bufs = ((sv0, dv0, iv0, ib0, rv0, wv0, semi0, semg0, semw0, dsc0, sems0),
                (sv1, dv1, iv1, ib1, rv1, wv1, semi1, semg1, semw1, dsc1, sems1))

        def ebase(g):
            return wid * EPW + g * CH

        def idx_copies(g, b):
            sv, dv, iv, semi = bufs[b][0], bufs[b][1], bufs[b][2], bufs[b][6]
            return (pltpu.make_async_copy(src_hbm.at[pl.ds(ebase(g), CH)], sv, semi),
                    pltpu.make_async_copy(dst_hbm.at[pl.ds(ebase(g), CH)], dv, semi),
                    pltpu.make_async_copy(knot_hbm.at[pl.ds(ebase(g), CH)], iv, semi))

        def add_tbl_offset(b):
            iv, ib = bufs[b][2], bufs[b][3]

            def off16(k, c2):
                sl = pl.ds(k * 16, 16)
                ib[sl] = iv[sl] + blk * TBLN
                return c2

            lax.fori_loop(0, CH // 16, off16, 0)

        def gw_copies(g, b):
            sv, ib, rv, wv, semg, semw = (bufs[b][0], bufs[b][3], bufs[b][4],
                                          bufs[b][5], bufs[b][7], bufs[b][8])
            return (pltpu.make_async_copy(hl_hbm.at[sv], rv, semg),
                    pltpu.make_async_copy(tbl_hbm.at[ib], wv, semw))

        def sc_copy(b):
            wv, dsc, sems = bufs[b][5], bufs[b][9], bufs[b][10]
            return pltpu.make_async_copy(wv, agg_sh.at[dsc], sems)

        z16 = jnp.zeros((16,), jnp.float32)

        def zrow(e, carry):
            for j in range(H // 16):
                wv0[e, pl.ds(j * 16, 16)] = z16
            return carry

        lax.fori_loop(0, CH, zrow, 0)

        rows_per_sub = NPAD // 16

        def zagg(g, carry):
            pltpu.sync_copy(wv0, agg_sh.at[pl.ds(s * rows_per_sub + g * CH, CH)])
            return carry

        lax.fori_loop(0, rows_per_sub // CH, zagg, 0)
        plsc.subcore_barrier()

        def phase(g, b, nxt_gather, nxt_idx, wait_prev_scatter=True):
            nb = 1 - b
            if nxt_gather:
                for cp in idx_copies(g + 1, nb):
                    cp.wait()
                add_tbl_offset(nb)
                if wait_prev_scatter:
                    sc_copy(nb).wait()
                for cp in gw_copies(g + 1, nb):
                    cp.start()
            for cp in gw_copies(g, b):
                cp.wait()

            rv, wv, dv, dsc = bufs[b][4], bufs[b][5], bufs[b][1], bufs[b][9]

            def mul(q, c2):
                for u in range(4):
                    e = q * 4 + u
                    for j in range(H // 16):
                        sl = pl.ds(j * 16, 16)
                        wv[e, sl] = wv[e, sl] * rv[e, sl]
                return c2

            lax.fori_loop(0, CH // 4, mul, 0)

            def dcopy(k, c2):
                sl = pl.ds(k * 16, 16)
                dsc[sl] = dv[sl]
                return c2

            lax.fori_loop(0, CH // 16, dcopy, 0)
            sc_copy(b).start(add=True)
            if nxt_idx:
                for cp in idx_copies(g + 2, b):
                    cp.start()

        for cp in idx_copies(0, 0) + idx_copies(1, 1):
            cp.start()
        for cp in idx_copies(0, 0):
            cp.wait()
        add_tbl_offset(0)
        for cp in gw_copies(0, 0):
            cp.start()

        phase(0, 0, True, True, wait_prev_scatter=False)
        phase(1, 1, True, True)

        def pair(k, carry):
            phase(2 * k + 2, 0, True, True)
            phase(2 * k + 3, 1, True, True)
            return carry

        lax.fori_loop(0, (NCH - 5) // 2, pair, 0)
        phase(NCH - 3, 0, True, True)
        phase(NCH - 2, 1, True, False)
        phase(NCH - 1, 0, False, False)
        sc_copy(1).wait()
        sc_copy(0).wait()
        plsc.subcore_barrier()

        def flush(g, carry):
            off = s * rows_per_sub + g * 128
            pltpu.sync_copy(agg_sh.at[pl.ds(off, 128)],
                            out_hbm.at[c, pl.ds(off, 128)])
            return carry

        lax.fori_loop(0, rows_per_sub // 128, flush, 0)

    return _sc_scatter


_sc_scatters = [_make_sc_scatter(i) for i in range(NB)]


def _silu(x):
    return x * (1.0 / (1.0 + jnp.exp(-x)))


def _knot_body(d2_ref, o_ref):
    t = jnp.minimum(jnp.sqrt(d2_ref[...]) * jnp.float32(TSCALE),
                    jnp.float32(TBLN - 2))
    o_ref[...] = (t + 0.5).astype(jnp.int32)


def _knot(d2):
    return pl.pallas_call(
        _knot_body,
        out_shape=jax.ShapeDtypeStruct((E // 128, 128), jnp.int32),
    )(d2.reshape(E // 128, 128)).reshape(E)


def _edge_mlp_body(d2_ref, cen_ref, w1_ref, b1_ref, w2_ref, b2_ref, out_ref):
    d = jnp.minimum(jnp.sqrt(d2_ref[...]), RCUT)
    rbf = jnp.exp(-GAMMA * (d - cen_ref[...]) ** 2)
    s1 = jnp.dot(rbf, w1_ref[0], preferred_element_type=jnp.float32) + b1_ref[0]
    s1 = _silu(s1)
    out_ref[0] = jnp.dot(s1, w2_ref[0], preferred_element_type=jnp.float32) + b2_ref[0]


def _edge_mlp(d2, centers, ew1, eb1, ew2, eb2):
    return pl.pallas_call(
        _edge_mlp_body,
        grid=(NB, TBLN // EBLK),
        in_specs=[
            pl.BlockSpec((EBLK, 1), lambda i, j: (j, 0)),
            pl.BlockSpec((1, R), lambda i, j: (0, 0)),
            pl.BlockSpec((1, R, H), lambda i, j: (i, 0, 0)),
            pl.BlockSpec((1, 1, H), lambda i, j: (i, 0, 0)),
            pl.BlockSpec((1, H, H), lambda i, j: (i, 0, 0)),
            pl.BlockSpec((1, 1, H), lambda i, j: (i, 0, 0)),
        ],
        out_specs=pl.BlockSpec((1, EBLK, H), lambda i, j: (i, j, 0)),
        out_shape=jax.ShapeDtypeStruct((NB, TBLN, H), jnp.float32),
    )(d2, centers, ew1, eb1, ew2, eb2)


def _matmul_body(x_ref, w_ref, o_ref):
    o_ref[...] = jnp.dot(x_ref[...], w_ref[...], preferred_element_type=jnp.float32)


def _matmul(x, w):
    return pl.pallas_call(
        _matmul_body,
        grid=(NPAD // NBLK,),
        in_specs=[
            pl.BlockSpec((NBLK, H), lambda i: (i, 0)),
            pl.BlockSpec((H, H), lambda i: (0, 0)),
        ],
        out_specs=pl.BlockSpec((NBLK, H), lambda i: (i, 0)),
        out_shape=jax.ShapeDtypeStruct((NPAD, H), jnp.float32),
    )(x, w)


def _node_body(h_ref, a_ref, nw1_ref, nb1_ref, nw2_ref, nb2_ref, g_ref, b_ref,
               lw_ref, ho_ref, hlo_ref):
    agg = a_ref[0] + a_ref[1]
    t = _silu(jnp.dot(agg, nw1_ref[...], preferred_element_type=jnp.float32)
              + nb1_ref[...])
    out = jnp.dot(t, nw2_ref[...], preferred_element_type=jnp.float32) + nb2_ref[...]
    hn = h_ref[...] + out
    mu = jnp.mean(hn, axis=-1, keepdims=True)
    xc = hn - mu
    var = jnp.mean(xc * xc, axis=-1, keepdims=True)
    hln = xc / jnp.sqrt(var + 1e-5) * g_ref[...] + b_ref[...]
    ho_ref[...] = hln
    hlo_ref[...] = jnp.dot(hln, lw_ref[...], preferred_element_type=jnp.float32)


def _node_update(h, aggp, nw1, nb1, nw2, nb2, g, b, lw_next):
    return pl.pallas_call(
        _node_body,
        grid=(NPAD // NBLK,),
        in_specs=[
            pl.BlockSpec((NBLK, H), lambda i: (i, 0)),
            pl.BlockSpec((2, NBLK, H), lambda i: (0, i, 0)),
            pl.BlockSpec((H, H), lambda i: (0, 0)),
            pl.BlockSpec((1, H), lambda i: (0, 0)),
            pl.BlockSpec((H, H), lambda i: (0, 0)),
            pl.BlockSpec((1, H), lambda i: (0, 0)),
            pl.BlockSpec((1, H), lambda i: (0, 0)),
            pl.BlockSpec((1, H), lambda i: (0, 0)),
            pl.BlockSpec((H, H), lambda i: (0, 0)),
        ],
        out_specs=[
            pl.BlockSpec((NBLK, H), lambda i: (i, 0)),
            pl.BlockSpec((NBLK, H), lambda i: (i, 0)),
        ],
        out_shape=[
            jax.ShapeDtypeStruct((NPAD, H), jnp.float32),
            jax.ShapeDtypeStruct((NPAD, H), jnp.float32),
        ],
    )(h, aggp, nw1, nb1, nw2, nb2, g, b, lw_next)


def _readout_body(h_ref, rw1_ref, rb1_ref, rw2t_ref, rb2_ref, o_ref):
    i = pl.program_id(0)
    e1 = jnp.dot(_silu(h_ref[...]), rw1_ref[...],
                 preferred_element_type=jnp.float32) + rb1_ref[...]
    e2 = _silu(e1)
    contrib = e2 * rw2t_ref[...]
    rows = i * NBLK + lax.broadcasted_iota(jnp.int32, (NBLK, R), 0)
    contrib = jnp.where(rows < N, contrib, 0.0)

    @pl.when(i == 0)
    def _():
        o_ref[...] = jnp.float32(N) * rb2_ref[...]

    o_ref[...] += jnp.sum(contrib)


def _readout(h, rw1, rb1, rw2t, rb2):
    return pl.pallas_call(
        _readout_body,
        grid=(NPAD // NBLK,),
        in_specs=[
            pl.BlockSpec((NBLK, H), lambda i: (i, 0)),
            pl.BlockSpec((H, R), lambda i: (0, 0)),
            pl.BlockSpec((1, R), lambda i: (0, 0)),
            pl.BlockSpec((1, R), lambda i: (0, 0)),
            pl.BlockSpec((1, 1), lambda i: (0, 0)),
        ],
        out_specs=pl.BlockSpec((1, 1), lambda i: (0, 0)),
        out_shape=jax.ShapeDtypeStruct((1, 1), jnp.float32),
    )(h, rw1, rb1, rw2t, rb2)


def kernel(Z, pos, edge_index, embed_W, edge_w1, edge_b1, edge_w2, edge_b2,
           lin_w, node_w1, node_b1, node_w2, node_b2, ln_g, ln_b,
           r_w1, r_b1, r_w2, r_b2):
    src = edge_index[0].astype(jnp.int32)
    dst = edge_index[1].astype(jnp.int32)
    z_pad = jnp.pad(Z.astype(jnp.int32), (0, NPAD - N))
    posf = jnp.pad(pos, ((0, 0), (0, H - 3)))

    h0, d2 = _sc_prep(z_pad, src, dst, posf, embed_W)
    knot = _knot(d2)

    dk = jnp.minimum(jnp.arange(TBLN, dtype=jnp.float32), TBLN - 2) / TSCALE
    centers = jnp.linspace(0.0, RCUT, R).astype(jnp.float32).reshape(1, R)
    tbl = _edge_mlp((dk * dk).reshape(TBLN, 1), centers, edge_w1,
                    edge_b1.reshape(NB, 1, H), edge_w2,
                    edge_b2.reshape(NB, 1, H))
    tbl2d = tbl.reshape(NB * TBLN, H)

    h = h0
    hl = _matmul(h, lin_w[0])
    for i in range(NB):
        aggp = _sc_scatters[i](src, dst, knot, tbl2d, hl)
        h, hl = _node_update(
            h, aggp, node_w1[i], node_b1[i].reshape(1, H),
            node_w2[i], node_b2[i].reshape(1, H),
            ln_g[i].reshape(1, H), ln_b[i].reshape(1, H),
            lin_w[(i + 1) % NB])

    total = _readout(h, r_w1, r_b1.reshape(1, R), r_w2.reshape(1, R),
                     r_b2.reshape(1, 1))
    return total[0, 0]


# speedup vs baseline: 1.0334x; 1.0334x over previous
import functools

import jax
import jax.numpy as jnp
from jax import lax
from jax.experimental import pallas as pl
from jax.experimental.pallas import tpu as pltpu
from jax.experimental.pallas import tpu_sc as plsc

N = 10000
NPAD = 10240
E = 320000
H = 128
R = 64
NB = 4
RCUT = 6.0
GAMMA = 10.0 / (RCUT * RCUT)

NW = 32
EPW = E // NW
CH = 80
NCH = EPW // CH
EBLK = 2048
TBLN = 32768
TSCALE = (TBLN - 2) / RCUT
NBLK = 1024

_mesh = plsc.VectorSubcoreMesh(core_axis_name="c", subcore_axis_name="s")


@functools.partial(
    pl.kernel,
    mesh=_mesh,
    compiler_params=pltpu.CompilerParams(use_tc_tiling_on_sc=False),
    out_type=[
        jax.ShapeDtypeStruct((NPAD, H), jnp.float32),
        jax.ShapeDtypeStruct((E,), jnp.float32),
    ],
    scratch_types=[
        pltpu.VMEM((64,), jnp.int32),
        pltpu.VMEM((64, H), jnp.float32),
        pltpu.VMEM((CH,), jnp.int32),
        pltpu.VMEM((CH,), jnp.int32),
        pltpu.VMEM((CH, 16), jnp.float32),
        pltpu.VMEM((CH, 16), jnp.float32),
        pltpu.VMEM((CH,), jnp.int32),
        pltpu.VMEM((CH,), jnp.int32),
        pltpu.VMEM((CH, 16), jnp.float32),
        pltpu.VMEM((CH, 16), jnp.float32),
        pltpu.VMEM((EPW,), jnp.float32),
        pltpu.SemaphoreType.DMA,
        pltpu.SemaphoreType.DMA,
        pltpu.SemaphoreType.DMA,
    ],
)
def _sc_prep(z_hbm, src_hbm, dst_hbm, posf_hbm, emb_hbm,
             h0_hbm, knot_hbm,
             zv, rowsv, sv0, dv0, prs0, prd0, sv1, dv1, prs1, prd1,
             kall, sem, semi0, semi1):
    c = lax.axis_index("c")
    s = lax.axis_index("s")
    wid = c * 16 + s
    bufs = ((sv0, dv0, prs0, prd0, semi0),
            (sv1, dv1, prs1, prd1, semi1))

    def emb_chunk(g, carry):
        rb = wid * (NPAD // NW) + g * 64
        pltpu.sync_copy(z_hbm.at[pl.ds(rb, 64)], zv)
        pltpu.async_copy(emb_hbm.at[zv], rowsv, sem).wait()
        pltpu.sync_copy(rowsv, h0_hbm.at[pl.ds(rb, 64)])
        return carry

    lax.fori_loop(0, (NPAD // NW) // 64, emb_chunk, 0)

    def idx_copies(g, b):
        sv, dv, semi = bufs[b][0], bufs[b][1], bufs[b][4]
        base = wid * EPW + g * CH
        return (pltpu.make_async_copy(src_hbm.at[pl.ds(base, CH)], sv, semi),
                pltpu.make_async_copy(dst_hbm.at[pl.ds(base, CH)], dv, semi))

    def pos_copies(b):
        sv, dv, prs, prd = bufs[b][0], bufs[b][1], bufs[b][2], bufs[b][3]
        return (pltpu.make_async_copy(posf_hbm.at[sv], prs, sem),
                pltpu.make_async_copy(posf_hbm.at[dv], prd, sem))

    lane = lax.iota(jnp.int32, 16)

    def phase(g, b, nxt_gather, nxt_idx):
        nb = 1 - b
        if nxt_gather:
            for cp in idx_copies(g + 1, nb):
                cp.wait()
            for cp in pos_copies(nb):
                cp.start()
        for cp in pos_copies(b):
            cp.wait()
        prs, prd = bufs[b][2], bufs[b][3]

        def dist16(k, c2):
            acc = jnp.zeros((16,), jnp.float32)
            for j in range(16):
                e = k * 16 + j
                df = prs[e, pl.ds(0, 16)] - prd[e, pl.ds(0, 16)]
                sq = df * df
                acc = jnp.where(lane == j, sq[0] + sq[1] + sq[2], acc)
            kall[pl.ds(g * CH + k * 16, 16)] = acc
            return c2

        lax.fori_loop(0, CH // 16, dist16, 0)
        if nxt_idx:
            for cp in idx_copies(g + 2, b):
                cp.start()

    for cp in idx_copies(0, 0) + idx_copies(1, 1):
        cp.start()
    for cp in idx_copies(0, 0):
        cp.wait()
    for cp in pos_copies(0):
        cp.start()

    def pair(k, carry):
        phase(2 * k, 0, True, True)
        phase(2 * k + 1, 1, True, True)
        return carry

    lax.fori_loop(0, (NCH - 3) // 2, pair, 0)
    phase(NCH - 3, 0, True, True)
    phase(NCH - 2, 1, True, False)
    phase(NCH - 1, 0, False, False)
    pltpu.sync_copy(kall, knot_hbm.at[pl.ds(wid * EPW, EPW)])


def _make_sc_scatter(blk):
    @functools.partial(
        pl.kernel,
        mesh=_mesh,
        out_type=jax.ShapeDtypeStruct((2, NPAD, H), jnp.float32),
        scratch_types=[
            pltpu.VMEM((CH,), jnp.int32),
            pltpu.VMEM((CH,), jnp.int32),
            pltpu.VMEM((CH,), jnp.int32),
            pltpu.VMEM((CH,), jnp.int32),
            pltpu.VMEM((CH, H), jnp.float32),
            pltpu.VMEM((CH, H), jnp.float32),
            pltpu.VMEM((CH,), jnp.int32),
            pltpu.VMEM((CH,), jnp.int32),
            pltpu.VMEM((CH,), jnp.int32),
            pltpu.VMEM((CH,), jnp.int32),
            pltpu.VMEM((CH, H), jnp.float32),
            pltpu.VMEM((CH, H), jnp.float32),
            pltpu.VMEM((CH,), jnp.int32),
            pltpu.VMEM((CH,), jnp.int32),
            pltpu.VMEM_SHARED((NPAD, H), jnp.float32),
            pltpu.SemaphoreType.DMA,
            pltpu.SemaphoreType.DMA,
            pltpu.SemaphoreType.DMA,
            pltpu.SemaphoreType.DMA,
            pltpu.SemaphoreType.DMA,
            pltpu.SemaphoreType.DMA,
            pltpu.SemaphoreType.DMA,
            pltpu.SemaphoreType.DMA,
        ],
    )
    def _sc_scatter(src_hbm, dst_hbm, knot_hbm, tbl_hbm, hl_hbm,
                    out_hbm,
                    sv0, dv0, iv0, ib0, rv0, wv0,
                    sv1, dv1, iv1, ib1, rv1, wv1, dsc0, dsc1, agg_sh,
                    semi0, semg0, semw0, sems0, semi1, semg1, semw1, sems1):
        c = lax.axis_index("c")
        s = lax.axis_index("s")
        wid = c * 16 + s
        bufs = ((sv0, dv0, iv0, ib0, rv0, wv0, semi0, semg0, semw0, dsc0, sems0),
                (sv1, dv1, iv1, ib1, rv1, wv1, semi1, semg1, semw1, dsc1, sems1))

        def ebase(g):
            return wid * EPW + g * CH

        def idx_copies(g, b):
            sv, dv, iv, semi = bufs[b][0], bufs[b][1], bufs[b][2], bufs[b][6]
            return (pltpu.make_async_copy(src_hbm.at[pl.ds(ebase(g), CH)], sv, semi),
                    pltpu.make_async_copy(dst_hbm.at[pl.ds(ebase(g), CH)], dv, semi),
                    pltpu.make_async_copy(knot_hbm.at[pl.ds(ebase(g), CH)], iv, semi))

        def add_tbl_offset(b):
            iv, ib = bufs[b][2], bufs[b][3]

            def off16(k, c2):
                sl = pl.ds(k * 16, 16)
                ib[sl] = iv[sl] + blk * TBLN
                return c2

            lax.fori_loop(0, CH // 16, off16, 0)

        def gw_copies(g, b):
            sv, ib, rv, wv, semg, semw = (bufs[b][0], bufs[b][3], bufs[b][4],
                                          bufs[b][5], bufs[b][7], bufs[b][8])
            return (pltpu.make_async_copy(hl_hbm.at[sv], rv, semg),
                    pltpu.make_async_copy(tbl_hbm.at[ib], wv, semw))

        def sc_copy(b):
            wv, dsc, sems = bufs[b][5], bufs[b][9], bufs[b][10]
            return pltpu.make_async_copy(wv, agg_sh.at[dsc], sems)

        z16 = jnp.zeros((16,), jnp.float32)

        def zrow(e, carry):
            for j in range(H // 16):
                wv0[e, pl.ds(j * 16, 16)] = z16
            return carry

        lax.fori_loop(0, CH, zrow, 0)

        rows_per_sub = NPAD // 16

        def zagg(g, carry):
            pltpu.sync_copy(wv0, agg_sh.at[pl.ds(s * rows_per_sub + g * CH, CH)])
            return carry

        lax.fori_loop(0, rows_per_sub // CH, zagg, 0)
        plsc.subcore_barrier()

        def phase(g, b, nxt_gather, nxt_idx, wait_prev_scatter=True):
            nb = 1 - b
            if nxt_gather:
                for cp in idx_copies(g + 1, nb):
                    cp.wait()
                add_tbl_offset(nb)
                if wait_prev_scatter:
                    sc_copy(nb).wait()
                for cp in gw_copies(g + 1, nb):
                    cp.start()
            for cp in gw_copies(g, b):
                cp.wait()

            rv, wv, dv, dsc = bufs[b][4], bufs[b][5], bufs[b][1], bufs[b][9]

            def mul(q, c2):
                for u in range(4):
                    e = q * 4 + u
                    for j in range(H // 16):
                        sl = pl.ds(j * 16, 16)
                        wv[e, sl] = wv[e, sl] * rv[e, sl]
                return c2

            lax.fori_loop(0, CH // 4, mul, 0)

            def dcopy(k, c2):
                sl = pl.ds(k * 16, 16)
                dsc[sl] = dv[sl]
                return c2

            lax.fori_loop(0, CH // 16, dcopy, 0)
            sc_copy(b).start(add=True)
            if nxt_idx:
                for cp in idx_copies(g + 2, b):
                    cp.start()

        for cp in idx_copies(0, 0) + idx_copies(1, 1):
            cp.start()
        for cp in idx_copies(0, 0):
            cp.wait()
        add_tbl_offset(0)
        for cp in gw_copies(0, 0):
            cp.start()

        phase(0, 0, True, True, wait_prev_scatter=False)
        phase(1, 1, True, True)

        def pair(k, carry):
            phase(2 * k + 2, 0, True, True)
            phase(2 * k + 3, 1, True, True)
            return carry

        lax.fori_loop(0, (NCH - 5) // 2, pair, 0)
        phase(NCH - 3, 0, True, True)
        phase(NCH - 2, 1, True, False)
        phase(NCH - 1, 0, False, False)
        sc_copy(1).wait()
        sc_copy(0).wait()
        plsc.subcore_barrier()

        def flush(g, carry):
            off = s * rows_per_sub + g * 128
            pltpu.sync_copy(agg_sh.at[pl.ds(off, 128)],
                            out_hbm.at[c, pl.ds(off, 128)])
            return carry

        lax.fori_loop(0, rows_per_sub // 128, flush, 0)

    return _sc_scatter


_sc_scatters = [_make_sc_scatter(i) for i in range(NB)]


def _silu(x):
    return x * (1.0 / (1.0 + jnp.exp(-x)))


def _knot_body(d2_ref, o_ref):
    t = jnp.minimum(jnp.sqrt(d2_ref[...]) * jnp.float32(TSCALE),
                    jnp.float32(TBLN - 2))
    o_ref[...] = (t + 0.5).astype(jnp.int32)


def _knot(d2):
    return pl.pallas_call(
        _knot_body,
        out_shape=jax.ShapeDtypeStruct((E // 128, 128), jnp.int32),
    )(d2.reshape(E // 128, 128)).reshape(E)


def _edge_mlp_body(d2_ref, cen_ref, w1_ref, b1_ref, w2_ref, b2_ref, out_ref):
    d = jnp.minimum(jnp.sqrt(d2_ref[...]), RCUT)
    rbf = jnp.exp(-GAMMA * (d - cen_ref[...]) ** 2)
    s1 = jnp.dot(rbf, w1_ref[0], preferred_element_type=jnp.float32) + b1_ref[0]
    s1 = _silu(s1)
    out_ref[0] = jnp.dot(s1, w2_ref[0], preferred_element_type=jnp.float32) + b2_ref[0]


def _edge_mlp(d2, centers, ew1, eb1, ew2, eb2):
    return pl.pallas_call(
        _edge_mlp_body,
        grid=(NB, TBLN // EBLK),
        in_specs=[
            pl.BlockSpec((EBLK, 1), lambda i, j: (j, 0)),
            pl.BlockSpec((1, R), lambda i, j: (0, 0)),
            pl.BlockSpec((1, R, H), lambda i, j: (i, 0, 0)),
            pl.BlockSpec((1, 1, H), lambda i, j: (i, 0, 0)),
            pl.BlockSpec((1, H, H), lambda i, j: (i, 0, 0)),
            pl.BlockSpec((1, 1, H), lambda i, j: (i, 0, 0)),
        ],
        out_specs=pl.BlockSpec((1, EBLK, H), lambda i, j: (i, j, 0)),
        out_shape=jax.ShapeDtypeStruct((NB, TBLN, H), jnp.float32),
    )(d2, centers, ew1, eb1, ew2, eb2)


def _matmul_body(x_ref, w_ref, o_ref):
    o_ref[...] = jnp.dot(x_ref[...], w_ref[...], preferred_element_type=jnp.float32)


def _matmul(x, w):
    return pl.pallas_call(
        _matmul_body,
        grid=(NPAD // NBLK,),
        in_specs=[
            pl.BlockSpec((NBLK, H), lambda i: (i, 0)),
            pl.BlockSpec((H, H), lambda i: (0, 0)),
        ],
        out_specs=pl.BlockSpec((NBLK, H), lambda i: (i, 0)),
        out_shape=jax.ShapeDtypeStruct((NPAD, H), jnp.float32),
    )(x, w)


def _node_body(h_ref, a_ref, nw1_ref, nb1_ref, nw2_ref, nb2_ref, g_ref, b_ref,
               lw_ref, ho_ref, hlo_ref):
    agg = a_ref[0] + a_ref[1]
    t = _silu(jnp.dot(agg, nw1_ref[...], preferred_element_type=jnp.float32)
              + nb1_ref[...])
    out = jnp.dot(t, nw2_ref[...], preferred_element_type=jnp.float32) + nb2_ref[...]
    hn = h_ref[...] + out
    mu = jnp.mean(hn, axis=-1, keepdims=True)
    xc = hn - mu
    var = jnp.mean(xc * xc, axis=-1, keepdims=True)
    hln = xc / jnp.sqrt(var + 1e-5) * g_ref[...] + b_ref[...]
    ho_ref[...] = hln
    hlo_ref[...] = jnp.dot(hln, lw_ref[...], preferred_element_type=jnp.float32)


def _node_update(h, aggp, nw1, nb1, nw2, nb2, g, b, lw_next):
    return pl.pallas_call(
        _node_body,
        grid=(NPAD // NBLK,),
        in_specs=[
            pl.BlockSpec((NBLK, H), lambda i: (i, 0)),
            pl.BlockSpec((2, NBLK, H), lambda i: (0, i, 0)),
            pl.BlockSpec((H, H), lambda i: (0, 0)),
            pl.BlockSpec((1, H), lambda i: (0, 0)),
            pl.BlockSpec((H, H), lambda i: (0, 0)),
            pl.BlockSpec((1, H), lambda i: (0, 0)),
            pl.BlockSpec((1, H), lambda i: (0, 0)),
            pl.BlockSpec((1, H), lambda i: (0, 0)),
            pl.BlockSpec((H, H), lambda i: (0, 0)),
        ],
        out_specs=[
            pl.BlockSpec((NBLK, H), lambda i: (i, 0)),
            pl.BlockSpec((NBLK, H), lambda i: (i, 0)),
        ],
        out_shape=[
            jax.ShapeDtypeStruct((NPAD, H), jnp.float32),
            jax.ShapeDtypeStruct((NPAD, H), jnp.float32),
        ],
    )(h, aggp, nw1, nb1, nw2, nb2, g, b, lw_next)


def _readout_body(h_ref, rw1_ref, rb1_ref, rw2t_ref, rb2_ref, o_ref):
    i = pl.program_id(0)
    e1 = jnp.dot(_silu(h_ref[...]), rw1_ref[...],
                 preferred_element_type=jnp.float32) + rb1_ref[...]
    e2 = _silu(e1)
    contrib = e2 * rw2t_ref[...]
    rows = i * NBLK + lax.broadcasted_iota(jnp.int32, (NBLK, R), 0)
    contrib = jnp.where(rows < N, contrib, 0.0)

    @pl.when(i == 0)
    def _():
        o_ref[...] = jnp.float32(N) * rb2_ref[...]

    o_ref[...] += jnp.sum(contrib)


def _readout(h, rw1, rb1, rw2t, rb2):
    return pl.pallas_call(
        _readout_body,
        grid=(NPAD // NBLK,),
        in_specs=[
            pl.BlockSpec((NBLK, H), lambda i: (i, 0)),
            pl.BlockSpec((H, R), lambda i: (0, 0)),
            pl.BlockSpec((1, R), lambda i: (0, 0)),
            pl.BlockSpec((1, R), lambda i: (0, 0)),
            pl.BlockSpec((1, 1), lambda i: (0, 0)),
        ],
        out_specs=pl.BlockSpec((1, 1), lambda i: (0, 0)),
        out_shape=jax.ShapeDtypeStruct((1, 1), jnp.float32),
    )(h, rw1, rb1, rw2t, rb2)


def kernel(Z, pos, edge_index, embed_W, edge_w1, edge_b1, edge_w2, edge_b2,
           lin_w, node_w1, node_b1, node_w2, node_b2, ln_g, ln_b,
           r_w1, r_b1, r_w2, r_b2):
    src = edge_index[0].astype(jnp.int32)
    dst = edge_index[1].astype(jnp.int32)
    z_pad = jnp.pad(Z.astype(jnp.int32), (0, NPAD - N))
    posf = jnp.pad(pos, ((0, 0), (0, 13)))

    h0, d2 = _sc_prep(z_pad, src, dst, posf, embed_W)
    knot = _knot(d2)

    dk = jnp.minimum(jnp.arange(TBLN, dtype=jnp.float32), TBLN - 2) / TSCALE
    centers = jnp.linspace(0.0, RCUT, R).astype(jnp.float32).reshape(1, R)
    tbl = _edge_mlp((dk * dk).reshape(TBLN, 1), centers, edge_w1,
                    edge_b1.reshape(NB, 1, H), edge_w2,
                    edge_b2.reshape(NB, 1, H))
    tbl2d = tbl.reshape(NB * TBLN, H)

    h = h0
    hl = _matmul(h, lin_w[0])
    for i in range(NB):
        aggp = _sc_scatters[i](src, dst, knot, tbl2d, hl)
        h, hl = _node_update(
            h, aggp, node_w1[i], node_b1[i].reshape(1, H),
            node_w2[i], node_b2[i].reshape(1, H),
            ln_g[i].reshape(1, H), ln_b[i].reshape(1, H),
            lin_w[(i + 1) % NB])

    total = _readout(h, r_w1, r_b1.reshape(1, R), r_w2.reshape(1, R),
                     r_b2.reshape(1, 1))
    return total[0, 0]
